# Initial kernel scaffold; baseline (speedup 1.0000x reference)
import jax
import jax.numpy as jnp
from jax.experimental import pallas as pl


def kernel(x_a, x_b, edge_index_ab, edge_index_ba, W_lin_a, b_lin_a, W_rot_a, b_rot_a, W_lin_b, b_lin_b, W_rot_b, b_rot_b, W_res, b_res, gamma1, beta1, gamma2, beta2):
    raise NotImplementedError("write your pallas kernel here")


# speedup vs baseline: 1.2586x; 1.2586x over previous
import functools

import jax
import jax.numpy as jnp
from jax import lax
from jax.experimental import pallas as pl
from jax.experimental.pallas import tpu as pltpu
from jax.experimental.pallas import tpu_sc as plsc

NA = 10000
D = 512
E = 80000
NCH = 4
CW = D // NCH
NC = 2
NS = 16
NW = NC * NS
EPAD = 81920
EPW = EPAD // NW
SUB = 128
NSUB = EPW // SUB
RPT = 640
NPAD = NS * RPT
TRASH = NA

RB = 10
BR = NA // RB


def _sc_aggregate(xflat, src_idx, dst_idx, zeros):
    mesh = plsc.VectorSubcoreMesh(core_axis_name="c", subcore_axis_name="s")

    @functools.partial(
        pl.kernel,
        mesh=mesh,
        out_type=jax.ShapeDtypeStruct((NCH * NC * NPAD, CW), jnp.float32),
        scratch_types=[
            pltpu.VMEM((NSUB, SUB), jnp.int32),
            pltpu.VMEM((NSUB, SUB), jnp.int32),
            pltpu.VMEM((SUB, CW), jnp.float32),
            pltpu.VMEM_SHARED((NPAD, CW), jnp.float32),
        ],
    )
    def k(x_hbm, src_hbm, dst_hbm, zeros_hbm, out_hbm,
          src_v, dst_v, rows_v, agg):
        core = lax.axis_index("c")
        sid = lax.axis_index("s")
        wid = core * NS + sid
        my_rows = sid * RPT
        pltpu.sync_copy(dst_hbm.at[wid], dst_v)
        for c in range(NCH):
            pltpu.sync_copy(zeros_hbm, agg.at[pl.ds(my_rows, RPT), :])
            pltpu.sync_copy(src_hbm.at[c, wid], src_v)
            plsc.subcore_barrier()

            def body(j, carry):
                pltpu.sync_copy(x_hbm.at[src_v.at[j]], rows_v)
                pltpu.sync_copy(rows_v, agg.at[dst_v.at[j]], add=True)
                return carry

            lax.fori_loop(0, NSUB, body, 0)
            plsc.subcore_barrier()
            out_base = (c * NC + core) * NPAD + my_rows
            pltpu.sync_copy(agg.at[pl.ds(my_rows, RPT), :],
                            out_hbm.at[pl.ds(out_base, RPT), :])

    return k(xflat, src_idx, dst_idx, zeros)


def _dense_body(agg_ref, x_ref, wl_ref, wrot_ref, wres_ref, b1_ref, b2_ref,
                out_ref, stats_ref):
    i = pl.program_id(0)
    c = pl.program_id(1)
    part = agg_ref[0, 0] + agg_ref[0, 1]
    contrib = jnp.dot(part, wl_ref[0], preferred_element_type=jnp.float32)

    @pl.when(c == 0)
    def _init():
        out_ref[...] = contrib

    @pl.when(c > 0)
    def _acc():
        out_ref[...] += contrib

    @pl.when(c == NCH - 1)
    def _finish():
        xb = x_ref[...]
        lin = (out_ref[...]
               + jnp.dot(xb, wrot_ref[...], preferred_element_type=jnp.float32)
               + b1_ref[...])
        res = jnp.maximum(
            jnp.dot(xb, wres_ref[...], preferred_element_type=jnp.float32)
            + b2_ref[...], 0.0)
        pre = jnp.maximum(lin, 0.0) + res
        out_ref[...] = pre
        s1 = jnp.sum(pre, axis=0, keepdims=True)
        s2 = jnp.sum(pre * pre, axis=0, keepdims=True)
        st = jnp.concatenate([s1, s2, jnp.zeros((6, D), jnp.float32)], axis=0)

        @pl.when(i == 0)
        def _st_init():
            stats_ref[...] = st

        @pl.when(i > 0)
        def _st_acc():
            stats_ref[...] += st


def _dense(agg, x, wl4, wrot_t, wres_t, b1, b2):
    return pl.pallas_call(
        _dense_body,
        grid=(RB, NCH),
        in_specs=[
            pl.BlockSpec((1, NC, BR, CW), lambda i, c: (c, 0, i, 0)),
            pl.BlockSpec((BR, D), lambda i, c: (i, 0)),
            pl.BlockSpec((1, CW, D), lambda i, c: (c, 0, 0)),
            pl.BlockSpec((D, D), lambda i, c: (0, 0)),
            pl.BlockSpec((D, D), lambda i, c: (0, 0)),
            pl.BlockSpec((1, D), lambda i, c: (0, 0)),
            pl.BlockSpec((1, D), lambda i, c: (0, 0)),
        ],
        out_specs=[
            pl.BlockSpec((BR, D), lambda i, c: (i, 0)),
            pl.BlockSpec((8, D), lambda i, c: (0, 0)),
        ],
        out_shape=[
            jax.ShapeDtypeStruct((NA, D), jnp.float32),
            jax.ShapeDtypeStruct((8, D), jnp.float32),
        ],
    )(agg, x, wl4, wrot_t, wres_t, b1, b2)


def _bn_body(pre_ref, st_ref, g_ref, b_ref, out_ref):
    mean = st_ref[0:1, :] * (1.0 / NA)
    var = st_ref[1:2, :] * (1.0 / NA) - mean * mean
    inv = lax.rsqrt(var + 1e-5)
    out_ref[...] = (pre_ref[...] - mean) * inv * g_ref[...] + b_ref[...]


def _bn(pre, stats, gamma, beta):
    return pl.pallas_call(
        _bn_body,
        grid=(RB,),
        in_specs=[
            pl.BlockSpec((BR, D), lambda i: (i, 0)),
            pl.BlockSpec((8, D), lambda i: (0, 0)),
            pl.BlockSpec((1, D), lambda i: (0, 0)),
            pl.BlockSpec((1, D), lambda i: (0, 0)),
        ],
        out_specs=pl.BlockSpec((BR, D), lambda i: (i, 0)),
        out_shape=jax.ShapeDtypeStruct((NA, D), jnp.float32),
    )(pre, stats, gamma, beta)


def _prep_edges(ei):
    src = ei[0].astype(jnp.int32)
    dst = ei[1].astype(jnp.int32)
    pad = EPAD - E
    src = jnp.concatenate([src, jnp.zeros((pad,), jnp.int32)])
    dst = jnp.concatenate([dst, jnp.full((pad,), TRASH, jnp.int32)])
    offs = (jnp.arange(NCH, dtype=jnp.int32) * NA)[:, None]
    src_idx = (src[None, :] + offs).reshape(NCH, NW, NSUB, SUB)
    dst_idx = dst.reshape(NW, NSUB, SUB)
    return src_idx, dst_idx


def _chunked(x):
    return x.reshape(NA, NCH, CW).transpose(1, 0, 2).reshape(NCH * NA, CW)


def kernel(x_a, x_b, edge_index_ab, edge_index_ba,
           W_lin_a, b_lin_a, W_rot_a, b_rot_a,
           W_lin_b, b_lin_b, W_rot_b, b_rot_b,
           W_res, b_res, gamma1, beta1, gamma2, beta2):
    zeros = jnp.zeros((RPT, CW), jnp.float32)
    src_ab, dst_ab = _prep_edges(edge_index_ab)
    src_ba, dst_ba = _prep_edges(edge_index_ba)

    aggb_raw = _sc_aggregate(_chunked(x_a), src_ab, dst_ab, zeros)
    agga_raw = _sc_aggregate(_chunked(x_b), src_ba, dst_ba, zeros)
    aggb = aggb_raw.reshape(NCH, NC, NPAD, CW)[:, :, :NA, :]
    agga = agga_raw.reshape(NCH, NC, NPAD, CW)[:, :, :NA, :]

    wres_t = W_res.T
    bres = b_res.reshape(1, D)
    pre_a, st_a = _dense(agga, x_a, W_lin_b.T.reshape(NCH, CW, D), W_rot_b.T,
                         wres_t, (b_lin_b + b_rot_b).reshape(1, D), bres)
    pre_b, st_b = _dense(aggb, x_b, W_lin_a.T.reshape(NCH, CW, D), W_rot_a.T,
                         wres_t, (b_lin_a + b_rot_a).reshape(1, D), bres)

    out_a = _bn(pre_a, st_a, gamma1.reshape(1, D), beta1.reshape(1, D))
    out_b = _bn(pre_b, st_b, gamma2.reshape(1, D), beta2.reshape(1, D))
    return (out_a, out_b)
